# baseline (speedup 1.0000x reference)
import functools

import jax
import jax.numpy as jnp
from jax import lax
from jax.experimental import pallas as pl
from jax.experimental.pallas import tpu as pltpu
from jax.experimental.pallas import tpu_sc as plsc

HID = 2048
EPS = 1e-05

NUM_CORES = 2
NUM_SUBCORES = 16
NUM_WORKERS = NUM_CORES * NUM_SUBCORES
LANES = 16
VECS_PER_ROW = HID // LANES

CHUNK = 8
NBUF = 2


def _dyn_gather(x, idx):
    dnums = lax.GatherDimensionNumbers(
        offset_dims=(), collapsed_slice_dims=(0,), start_index_map=(0,))
    return lax.gather(x, idx[:, None], dnums, (1,),
                      mode=lax.GatherScatterMode.PROMISE_IN_BOUNDS)


def _rsqrt_newton(x):
    i = lax.bitcast_convert_type(x, jnp.int32)
    i = jnp.int32(0x5F3759DF) - lax.shift_right_logical(i, 1)
    y = lax.bitcast_convert_type(i, jnp.float32)
    for _ in range(3):
        y = y * (1.5 - 0.5 * x * y * y)
    return y


def _make_fused(num_tokens):
    tok_per_w = num_tokens // NUM_WORKERS
    n_chunks = tok_per_w // CHUNK
    n_outer = n_chunks // NBUF
    inv_n = 1.0 / HID
    mesh = plsc.VectorSubcoreMesh(
        core_axis_name="c", subcore_axis_name="s")

    @functools.partial(
        pl.kernel,
        out_type=jax.ShapeDtypeStruct((num_tokens, HID), jnp.float32),
        mesh=mesh,
        scratch_types=[
            pltpu.VMEM((tok_per_w,), jnp.int32),
            pltpu.VMEM((tok_per_w,), jnp.int32),
            pltpu.VMEM((HID,), jnp.float32),
            pltpu.VMEM((HID,), jnp.float32),
            pltpu.VMEM((HID,), jnp.float32),
            pltpu.VMEM((NBUF, CHUNK, HID), jnp.float32),
            pltpu.VMEM((NBUF, CHUNK, HID), jnp.float32),
            pltpu.VMEM((NBUF, CHUNK, HID), jnp.float32),
            [pltpu.SemaphoreType.DMA] * NBUF,
            [pltpu.SemaphoreType.DMA] * NBUF,
            [pltpu.SemaphoreType.DMA] * NBUF,
        ],
    )
    def fused(ids_hbm, pids_hbm, wtab_hbm, ptab_hbm, trow_hbm, gamma_hbm,
              beta_hbm, out_hbm, idx_v, pidx_v, trow_v, gamma_v, beta_v,
              wbuf, pbuf, obuf, sem_w, sem_p, sem_o):
        wid = lax.axis_index("s") * NUM_CORES + lax.axis_index("c")
        base = wid * tok_per_w
        pltpu.sync_copy(ids_hbm.at[pl.ds(base, tok_per_w)], idx_v)
        pltpu.sync_copy(pids_hbm.at[pl.ds(base, tok_per_w)], pidx_v)
        pltpu.sync_copy(trow_hbm, trow_v)
        pltpu.sync_copy(gamma_hbm, gamma_v)
        pltpu.sync_copy(beta_hbm, beta_v)

        def fire_gathers(c, b):
            off = c * CHUNK
            pltpu.async_copy(
                wtab_hbm.at[idx_v.at[pl.ds(off, CHUNK)]], wbuf.at[b],
                sem_w[b])
            pltpu.async_copy(
                ptab_hbm.at[pidx_v.at[pl.ds(off, CHUNK)]], pbuf.at[b],
                sem_p[b])

        for b in range(NBUF):
            fire_gathers(b, b)

        def outer_body(o, carry):
            for b in range(NBUF):
                c = o * NBUF + b
                pltpu.make_async_copy(
                    wtab_hbm.at[idx_v.at[pl.ds(0, CHUNK)]], wbuf.at[b],
                    sem_w[b]).wait()
                pltpu.make_async_copy(
                    ptab_hbm.at[pidx_v.at[pl.ds(0, CHUNK)]], pbuf.at[b],
                    sem_p[b]).wait()
                @pl.when(o > 0)
                def _():
                    pltpu.make_async_copy(
                        obuf.at[b], out_hbm.at[pl.ds(0, CHUNK)],
                        sem_o[b]).wait()

                def v_body(v, accs):
                    sl = pl.ds(v * LANES, LANES)
                    tv = trow_v[sl]
                    ns = []
                    ns2 = []
                    for r in range(CHUNK):
                        e = wbuf[b, r, sl] + pbuf[b, r, sl] + tv
                        obuf[b, r, sl] = e
                        ns.append(accs[r] + e)
                        ns2.append(accs[CHUNK + r] + e * e)
                    return tuple(ns) + tuple(ns2)

                zero = jnp.zeros((LANES,), jnp.float32)
                accs = lax.fori_loop(
                    0, VECS_PER_ROW, v_body, (zero,) * (2 * CHUNK),
                    unroll=False)

                iota = lax.iota(jnp.int32, LANES)
                mus = []
                scales = []
                for r in range(CHUNK):
                    s = accs[r]
                    s2 = accs[CHUNK + r]
                    for k in (1, 2, 4, 8):
                        idx = iota ^ k
                        s = s + _dyn_gather(s, idx)
                        s2 = s2 + _dyn_gather(s2, idx)
                    mu = s * inv_n
                    var = s2 * inv_n - mu * mu
                    mus.append(mu)
                    scales.append(_rsqrt_newton(var + EPS))

                def v_body2(v, carry2):
                    sl = pl.ds(v * LANES, LANES)
                    gv = gamma_v[sl]
                    bv = beta_v[sl]
                    for r in range(CHUNK):
                        e = obuf[b, r, sl]
                        obuf[b, r, sl] = (e - mus[r]) * scales[r] * gv + bv
                    return carry2

                lax.fori_loop(0, VECS_PER_ROW, v_body2, 0, unroll=False)

                pltpu.async_copy(
                    obuf.at[b], out_hbm.at[pl.ds(base + c * CHUNK, CHUNK)],
                    sem_o[b])

                @pl.when(c + NBUF < n_chunks)
                def _():
                    fire_gathers(c + NBUF, b)
            return carry

        lax.fori_loop(0, n_outer, outer_body, 0, unroll=False)
        for b in range(NBUF):
            pltpu.make_async_copy(
                obuf.at[b], out_hbm.at[pl.ds(0, CHUNK)], sem_o[b]).wait()

    return fused


def kernel(input_ids, position_ids, word_table, pos_table, type_table,
           gamma, beta):
    b, s = input_ids.shape
    n = b * s
    out = _make_fused(n)(
        input_ids.reshape(n),
        position_ids.reshape(n),
        word_table,
        pos_table,
        type_table[0],
        gamma,
        beta,
    )
    return out.reshape(b, s, HID)


# speedup vs baseline: 1.1870x; 1.1870x over previous
import functools

import jax
import jax.numpy as jnp
from jax import lax
from jax.experimental import pallas as pl
from jax.experimental.pallas import tpu as pltpu
from jax.experimental.pallas import tpu_sc as plsc

HID = 2048
EPS = 1e-05

NUM_CORES = 2
NUM_SUBCORES = 16
NUM_WORKERS = NUM_CORES * NUM_SUBCORES
LANES = 16
VECS_PER_ROW = HID // LANES

CHUNK = 8
NBUF = 2


def _dyn_gather(x, idx):
    dnums = lax.GatherDimensionNumbers(
        offset_dims=(), collapsed_slice_dims=(0,), start_index_map=(0,))
    return lax.gather(x, idx[:, None], dnums, (1,),
                      mode=lax.GatherScatterMode.PROMISE_IN_BOUNDS)


def _rsqrt_newton(x):
    i = lax.bitcast_convert_type(x, jnp.int32)
    i = jnp.int32(0x5F3759DF) - lax.shift_right_logical(i, 1)
    y = lax.bitcast_convert_type(i, jnp.float32)
    for _ in range(3):
        y = y * (1.5 - 0.5 * x * y * y)
    return y


def _make_fused(num_tokens):
    tok_per_w = num_tokens // NUM_WORKERS
    n_chunks = tok_per_w // CHUNK
    n_outer = n_chunks // NBUF
    inv_n = 1.0 / HID
    mesh = plsc.VectorSubcoreMesh(
        core_axis_name="c", subcore_axis_name="s")

    @functools.partial(
        pl.kernel,
        out_type=jax.ShapeDtypeStruct((num_tokens, HID), jnp.float32),
        mesh=mesh,
        scratch_types=[
            pltpu.VMEM((tok_per_w,), jnp.int32),
            pltpu.VMEM((tok_per_w,), jnp.int32),
            pltpu.VMEM((HID,), jnp.float32),
            pltpu.VMEM((HID,), jnp.float32),
            pltpu.VMEM((HID,), jnp.float32),
            pltpu.VMEM((NBUF, CHUNK, HID), jnp.float32),
            pltpu.VMEM((NBUF, CHUNK, HID), jnp.float32),
            pltpu.VMEM((NBUF, CHUNK, HID), jnp.float32),
            [pltpu.SemaphoreType.DMA] * NBUF,
            [pltpu.SemaphoreType.DMA] * NBUF,
            [pltpu.SemaphoreType.DMA] * NBUF,
        ],
    )
    def fused(ids_hbm, pids_hbm, wtab_hbm, ptab_hbm, trow_hbm, gamma_hbm,
              beta_hbm, out_hbm, idx_v, pidx_v, trow_v, gamma_v, beta_v,
              wbuf, pbuf, obuf, sem_w, sem_p, sem_o):
        wid = lax.axis_index("s") * NUM_CORES + lax.axis_index("c")
        base = wid * tok_per_w
        pltpu.sync_copy(ids_hbm.at[pl.ds(base, tok_per_w)], idx_v)
        pltpu.sync_copy(pids_hbm.at[pl.ds(base, tok_per_w)], pidx_v)
        pltpu.sync_copy(trow_hbm, trow_v)
        pltpu.sync_copy(gamma_hbm, gamma_v)
        pltpu.sync_copy(beta_hbm, beta_v)

        def fire_gathers(c, b):
            off = c * CHUNK
            pltpu.async_copy(
                wtab_hbm.at[idx_v.at[pl.ds(off, CHUNK)]], wbuf.at[b],
                sem_w[b])
            pltpu.async_copy(
                ptab_hbm.at[pidx_v.at[pl.ds(off, CHUNK)]], pbuf.at[b],
                sem_p[b])

        for b in range(NBUF):
            fire_gathers(b, b)

        def outer_body(o, carry):
            for b in range(NBUF):
                c = o * NBUF + b
                pltpu.make_async_copy(
                    wtab_hbm.at[idx_v.at[pl.ds(0, CHUNK)]], wbuf.at[b],
                    sem_w[b]).wait()
                pltpu.make_async_copy(
                    ptab_hbm.at[pidx_v.at[pl.ds(0, CHUNK)]], pbuf.at[b],
                    sem_p[b]).wait()
                @pl.when(o > 0)
                def _():
                    pltpu.make_async_copy(
                        obuf.at[b], out_hbm.at[pl.ds(0, CHUNK)],
                        sem_o[b]).wait()

                def v_body(v, accs):
                    sl = pl.ds(v * LANES, LANES)
                    tv = trow_v[sl]
                    ns = []
                    ns2 = []
                    for r in range(CHUNK):
                        e = wbuf[b, r, sl] + pbuf[b, r, sl] + tv
                        obuf[b, r, sl] = e
                        ns.append(accs[r] + e)
                        ns2.append(accs[CHUNK + r] + e * e)
                    return tuple(ns) + tuple(ns2)

                zero = jnp.zeros((LANES,), jnp.float32)
                accs = plsc.parallel_loop(
                    0, VECS_PER_ROW, 1, unroll=4,
                    carry=(zero,) * (2 * CHUNK))(v_body)

                iota = lax.iota(jnp.int32, LANES)
                mus = []
                scales = []
                for r in range(CHUNK):
                    s = accs[r]
                    s2 = accs[CHUNK + r]
                    for k in (1, 2, 4, 8):
                        idx = iota ^ k
                        s = s + _dyn_gather(s, idx)
                        s2 = s2 + _dyn_gather(s2, idx)
                    mu = s * inv_n
                    var = s2 * inv_n - mu * mu
                    mus.append(mu)
                    scales.append(_rsqrt_newton(var + EPS))

                def v_body2(v):
                    sl = pl.ds(v * LANES, LANES)
                    gv = gamma_v[sl]
                    bv = beta_v[sl]
                    for r in range(CHUNK):
                        e = obuf[b, r, sl]
                        obuf[b, r, sl] = (e - mus[r]) * scales[r] * gv + bv

                plsc.parallel_loop(0, VECS_PER_ROW, 1, unroll=4)(v_body2)

                pltpu.async_copy(
                    obuf.at[b], out_hbm.at[pl.ds(base + c * CHUNK, CHUNK)],
                    sem_o[b])

                @pl.when(c + NBUF < n_chunks)
                def _():
                    fire_gathers(c + NBUF, b)
            return carry

        lax.fori_loop(0, n_outer, outer_body, 0, unroll=False)
        for b in range(NBUF):
            pltpu.make_async_copy(
                obuf.at[b], out_hbm.at[pl.ds(0, CHUNK)], sem_o[b]).wait()

    return fused


def kernel(input_ids, position_ids, word_table, pos_table, type_table,
           gamma, beta):
    b, s = input_ids.shape
    n = b * s
    out = _make_fused(n)(
        input_ids.reshape(n),
        position_ids.reshape(n),
        word_table,
        pos_table,
        type_table[0],
        gamma,
        beta,
    )
    return out.reshape(b, s, HID)


# speedup vs baseline: 1.2916x; 1.0881x over previous
import functools

import jax
import jax.numpy as jnp
from jax import lax
from jax.experimental import pallas as pl
from jax.experimental.pallas import tpu as pltpu
from jax.experimental.pallas import tpu_sc as plsc

HID = 2048
EPS = 1e-05

NUM_CORES = 2
NUM_SUBCORES = 16
NUM_WORKERS = NUM_CORES * NUM_SUBCORES
LANES = 16
VECS_PER_ROW = HID // LANES

CHUNK = 8
NBUF = 2


def _make_gather_sum(num_tokens):
    tok_per_w = num_tokens // NUM_WORKERS
    n_chunks = tok_per_w // CHUNK
    n_outer = n_chunks // NBUF
    mesh = plsc.VectorSubcoreMesh(
        core_axis_name="c", subcore_axis_name="s")

    @functools.partial(
        pl.kernel,
        out_type=jax.ShapeDtypeStruct((num_tokens, HID), jnp.float32),
        mesh=mesh,
        scratch_types=[
            pltpu.VMEM((tok_per_w,), jnp.int32),
            pltpu.VMEM((tok_per_w,), jnp.int32),
            pltpu.VMEM((NBUF, CHUNK, HID), jnp.float32),
            pltpu.VMEM((NBUF, CHUNK, HID), jnp.float32),
            pltpu.VMEM((NBUF, CHUNK, HID), jnp.float32),
            [pltpu.SemaphoreType.DMA] * NBUF,
            [pltpu.SemaphoreType.DMA] * NBUF,
            [pltpu.SemaphoreType.DMA] * NBUF,
        ],
    )
    def gather_sum(ids_hbm, pids_hbm, wtab_hbm, ptab_hbm, out_hbm,
                   idx_v, pidx_v, wbuf, pbuf, obuf, sem_w, sem_p, sem_o):
        wid = lax.axis_index("s") * NUM_CORES + lax.axis_index("c")
        base = wid * tok_per_w
        pltpu.sync_copy(ids_hbm.at[pl.ds(base, tok_per_w)], idx_v)
        pltpu.sync_copy(pids_hbm.at[pl.ds(base, tok_per_w)], pidx_v)

        def fire_gathers(c, b):
            off = c * CHUNK
            pltpu.async_copy(
                wtab_hbm.at[idx_v.at[pl.ds(off, CHUNK)]], wbuf.at[b],
                sem_w[b])
            pltpu.async_copy(
                ptab_hbm.at[pidx_v.at[pl.ds(off, CHUNK)]], pbuf.at[b],
                sem_p[b])

        for b in range(NBUF):
            fire_gathers(b, b)

        def outer_body(o, carry):
            for b in range(NBUF):
                c = o * NBUF + b
                pltpu.make_async_copy(
                    wtab_hbm.at[idx_v.at[pl.ds(0, CHUNK)]], wbuf.at[b],
                    sem_w[b]).wait()
                pltpu.make_async_copy(
                    ptab_hbm.at[pidx_v.at[pl.ds(0, CHUNK)]], pbuf.at[b],
                    sem_p[b]).wait()
                @pl.when(o > 0)
                def _():
                    pltpu.make_async_copy(
                        obuf.at[b], out_hbm.at[pl.ds(0, CHUNK)],
                        sem_o[b]).wait()

                def v_body(v):
                    sl = pl.ds(v * LANES, LANES)
                    for r in range(CHUNK):
                        obuf[b, r, sl] = wbuf[b, r, sl] + pbuf[b, r, sl]

                plsc.parallel_loop(0, VECS_PER_ROW, 1, unroll=4)(v_body)

                pltpu.async_copy(
                    obuf.at[b], out_hbm.at[pl.ds(base + c * CHUNK, CHUNK)],
                    sem_o[b])

                @pl.when(c + NBUF < n_chunks)
                def _():
                    fire_gathers(c + NBUF, b)
            return carry

        lax.fori_loop(0, n_outer, outer_body, 0, unroll=False)
        for b in range(NBUF):
            pltpu.make_async_copy(
                obuf.at[b], out_hbm.at[pl.ds(0, CHUNK)], sem_o[b]).wait()

    return gather_sum


def _ln_body(x_ref, t_ref, g_ref, b_ref, o_ref):
    e = x_ref[...] + t_ref[...]
    mu = jnp.mean(e, axis=-1, keepdims=True)
    d = e - mu
    var = jnp.mean(d * d, axis=-1, keepdims=True)
    o_ref[...] = d * lax.rsqrt(var + EPS) * g_ref[...] + b_ref[...]


def _layernorm(summed, type_row, gamma, beta, blk):
    n = summed.shape[0]
    return pl.pallas_call(
        _ln_body,
        grid=(n // blk,),
        in_specs=[
            pl.BlockSpec((blk, HID), lambda i: (i, 0)),
            pl.BlockSpec((1, HID), lambda i: (0, 0)),
            pl.BlockSpec((1, HID), lambda i: (0, 0)),
            pl.BlockSpec((1, HID), lambda i: (0, 0)),
        ],
        out_specs=pl.BlockSpec((blk, HID), lambda i: (i, 0)),
        out_shape=jax.ShapeDtypeStruct((n, HID), jnp.float32),
    )(summed, type_row, gamma, beta)


def kernel(input_ids, position_ids, word_table, pos_table, type_table,
           gamma, beta):
    b, s = input_ids.shape
    n = b * s
    summed = _make_gather_sum(n)(
        input_ids.reshape(n), position_ids.reshape(n), word_table, pos_table)
    out = _layernorm(
        summed,
        type_table[0:1, :],
        gamma.reshape(1, HID),
        beta.reshape(1, HID),
        blk=512,
    )
    return out.reshape(b, s, HID)
